# Initial kernel scaffold; baseline (speedup 1.0000x reference)
import jax
import jax.numpy as jnp
from jax.experimental import pallas as pl


def kernel(x, table):
    raise NotImplementedError("write your pallas kernel here")


# speedup vs baseline: 1.0247x; 1.0247x over previous
import functools

import jax
import jax.numpy as jnp
from jax import lax
from jax.experimental import pallas as pl
from jax.experimental.pallas import tpu as pltpu
from jax.experimental.pallas import tpu_sc as plsc

_NUM_BUCKETS = 100000
_HIDDEN = 128
_MAX_T = 0.7
_B = 16384

_NC = 2
_NS = 16
_L = 16
_NW = _NC * _NS
_BPW = _B // _NW
_CHUNK = 128
_NCHUNK = _BPW // _CHUNK

_mesh = plsc.VectorSubcoreMesh(core_axis_name="c", subcore_axis_name="s")


@functools.partial(
    pl.kernel,
    out_type=jax.ShapeDtypeStruct((_B, _HIDDEN), jnp.float32),
    mesh=_mesh,
    scratch_types=[
        pltpu.VMEM((_BPW,), jnp.float32),
        pltpu.VMEM((_NCHUNK, _CHUNK), jnp.int32),
        pltpu.VMEM((_BPW, _HIDDEN), jnp.float32),
        pltpu.SemaphoreType.DMA,
    ],
)
def _embed(x_hbm, table_hbm, out_hbm, x_v, idx_v, rows_v, sem):
    wid = lax.axis_index("s") * _NC + lax.axis_index("c")
    base = wid * _BPW

    pltpu.sync_copy(x_hbm.at[pl.ds(base, _BPW)], x_v)

    for j in range(_NCHUNK):
        for c in range(_CHUNK // _L):
            xv = x_v[pl.ds(j * _CHUNK + c * _L, _L)]
            t = jnp.minimum(jnp.maximum(xv / _MAX_T, 0.0), 1.0) * _NUM_BUCKETS
            x1 = t.astype(jnp.int32).astype(jnp.float32)
            v = (x1 / _MAX_T) * _NUM_BUCKETS
            idx = jnp.minimum(jnp.maximum(v, 0.0), float(_NUM_BUCKETS - 1))
            idx_v[j, pl.ds(c * _L, _L)] = idx.astype(jnp.int32)

    copies = [
        pltpu.async_copy(
            table_hbm.at[idx_v.at[j]],
            rows_v.at[pl.ds(j * _CHUNK, _CHUNK)],
            sem,
        )
        for j in range(_NCHUNK)
    ]
    for cp in copies:
        cp.wait()

    pltpu.sync_copy(rows_v, out_hbm.at[pl.ds(base, _BPW)])


def kernel(x, table):
    return _embed(x, table)


# speedup vs baseline: 23.7640x; 23.1916x over previous
import functools

import jax
import jax.numpy as jnp
from jax import lax
from jax.experimental import pallas as pl
from jax.experimental.pallas import tpu as pltpu
from jax.experimental.pallas import tpu_sc as plsc

_NUM_BUCKETS = 100000
_HIDDEN = 128
_MAX_T = 0.7
_B = 16384

_NC = 2
_NS = 16
_L = 16
_NW = _NC * _NS
_BPW = _B // _NW
_CHUNK = 128
_NCHUNK = _BPW // _CHUNK

_mesh = plsc.VectorSubcoreMesh(core_axis_name="c", subcore_axis_name="s")


@functools.partial(
    pl.kernel,
    out_type=jax.ShapeDtypeStruct((_B, _HIDDEN), jnp.float32),
    mesh=_mesh,
    scratch_types=[
        pltpu.VMEM((_BPW,), jnp.float32),
        pltpu.VMEM((_BPW,), jnp.int32),
        pltpu.VMEM((2, _HIDDEN), jnp.float32),
        pltpu.VMEM((_BPW, _HIDDEN), jnp.float32),
        pltpu.SemaphoreType.DMA,
    ],
)
def _embed(x_hbm, table_hbm, out_hbm, x_v, sel_v, r01_v, rows_v, sem):
    wid = lax.axis_index("s") * _NC + lax.axis_index("c")
    base = wid * _BPW

    pltpu.sync_copy(x_hbm.at[pl.ds(base, _BPW)], x_v)
    pltpu.sync_copy(table_hbm.at[pl.ds(0, 1)], r01_v.at[pl.ds(0, 1)])
    pltpu.sync_copy(table_hbm.at[pl.ds(_NUM_BUCKETS - 1, 1)],
                    r01_v.at[pl.ds(1, 1)])

    for i in range(_BPW // _L):
        xv = x_v[pl.ds(i * _L, _L)]
        t = jnp.minimum(jnp.maximum(xv / _MAX_T, 0.0), 1.0) * _NUM_BUCKETS
        sel_v[pl.ds(i * _L, _L)] = t.astype(jnp.int32)

    r0 = [r01_v[0, pl.ds(k * _L, _L)] for k in range(_HIDDEN // _L)]
    r1 = [r01_v[1, pl.ds(k * _L, _L)] for k in range(_HIDDEN // _L)]

    def body(g, carry):
        e0 = g * _L
        c16 = jnp.minimum(sel_v[pl.ds(e0, _L)], 1).astype(jnp.float32)
        for e in range(_L):
            c = lax.broadcast_in_dim(c16[e], (_L,), ())
            a = 1.0 - c
            for k in range(_HIDDEN // _L):
                rows_v[e0 + e, pl.ds(k * _L, _L)] = r0[k] * a + r1[k] * c
        return carry

    copies = []
    for j in range(_NCHUNK):
        lax.fori_loop(j * (_CHUNK // _L), (j + 1) * (_CHUNK // _L), body, 0)
        copies.append(pltpu.async_copy(
            rows_v.at[pl.ds(j * _CHUNK, _CHUNK)],
            out_hbm.at[pl.ds(base + j * _CHUNK, _CHUNK)],
            sem,
        ))
    for cp in copies:
        cp.wait()


def kernel(x, table):
    return _embed(x, table)


# speedup vs baseline: 24.6041x; 1.0354x over previous
import functools

import jax
import jax.numpy as jnp
from jax import lax
from jax.experimental import pallas as pl
from jax.experimental.pallas import tpu as pltpu
from jax.experimental.pallas import tpu_sc as plsc

_NUM_BUCKETS = 100000
_HIDDEN = 128
_MAX_T = 0.7
_B = 16384

_NC = 2
_NS = 16
_L = 16
_NW = _NC * _NS
_BPW = _B // _NW
_CHUNK = 128
_NCHUNK = _BPW // _CHUNK

_mesh = plsc.VectorSubcoreMesh(core_axis_name="c", subcore_axis_name="s")


@functools.partial(
    pl.kernel,
    out_type=jax.ShapeDtypeStruct((_B, _HIDDEN), jnp.float32),
    mesh=_mesh,
    scratch_types=[
        pltpu.VMEM((_BPW,), jnp.float32),
        pltpu.VMEM((2, _HIDDEN), jnp.float32),
        pltpu.VMEM((_BPW, _HIDDEN), jnp.float32),
        pltpu.SemaphoreType.DMA,
        pltpu.SemaphoreType.DMA,
    ],
)
def _embed(x_hbm, table_hbm, out_hbm, x_v, r01_v, rows_v, in_sem, out_sem):
    wid = lax.axis_index("s") * _NC + lax.axis_index("c")
    base = wid * _BPW

    in_copies = [
        pltpu.async_copy(x_hbm.at[pl.ds(base, _BPW)], x_v, in_sem),
        pltpu.async_copy(table_hbm.at[pl.ds(0, 1)], r01_v.at[pl.ds(0, 1)],
                         in_sem),
        pltpu.async_copy(table_hbm.at[pl.ds(_NUM_BUCKETS - 1, 1)],
                         r01_v.at[pl.ds(1, 1)], in_sem),
    ]
    for cp in in_copies:
        cp.wait()

    r0 = [r01_v[0, pl.ds(k * _L, _L)] for k in range(_HIDDEN // _L)]
    r1 = [r01_v[1, pl.ds(k * _L, _L)] for k in range(_HIDDEN // _L)]

    def body(g, carry):
        e0 = g * _L
        xv = x_v[pl.ds(e0, _L)]
        t = jnp.minimum(jnp.maximum(xv / _MAX_T, 0.0), 1.0) * _NUM_BUCKETS
        c16 = jnp.minimum(t.astype(jnp.int32), 1).astype(jnp.float32)
        for e in range(_L):
            c = lax.broadcast_in_dim(c16[e], (_L,), ())
            a = 1.0 - c
            for k in range(_HIDDEN // _L):
                rows_v[e0 + e, pl.ds(k * _L, _L)] = r0[k] * a + r1[k] * c
        return carry

    copies = []
    for j in range(_NCHUNK):
        lax.fori_loop(j * (_CHUNK // _L), (j + 1) * (_CHUNK // _L), body, 0)
        copies.append(pltpu.async_copy(
            rows_v.at[pl.ds(j * _CHUNK, _CHUNK)],
            out_hbm.at[pl.ds(base + j * _CHUNK, _CHUNK)],
            out_sem,
        ))
    for cp in copies:
        cp.wait()


def kernel(x, table):
    return _embed(x, table)
